# baseline (speedup 1.0000x reference)
import jax
import jax.numpy as jnp
from jax import lax
from jax.experimental import pallas as pl
from jax.experimental.pallas import tpu as pltpu
from jax.experimental.pallas import tpu_sc as plsc

NUM_SECTIONS = 4
NUM_CENTROIDS = 1024
EMBED_DIM = 256
SEC_DIM = EMBED_DIM // NUM_SECTIONS
COMMITMENT = 0.25
TM = 512
NW = 32
KAUG = 2 * SEC_DIM


def _dist_kernel(x_ref, cbm2_ref, cnc_ref, idxg_ref, loss_ref):
    i = pl.program_id(0)

    @pl.when(i == 0)
    def _():
        loss_ref[0, 0] = 0.0

    x_all = x_ref[...]
    scale = (1.0 + COMMITMENT) / (NUM_SECTIONS * SEC_DIM)
    loss_step = jnp.sum(x_all * x_all)
    for s in range(NUM_SECTIONS):
        x = x_all[:, s * SEC_DIM:(s + 1) * SEC_DIM]
        dT = jax.lax.dot_general(
            cbm2_ref[s], x, dimension_numbers=(((1,), (1,)), ((), ())),
            preferred_element_type=jnp.float32) + cnc_ref[s]
        dmin = jnp.min(dT, axis=0, keepdims=True)
        idx = jnp.argmin(dT, axis=0)
        idxg_ref[s, :] = idx + s * NUM_CENTROIDS
        loss_step += jnp.sum(dmin)
    loss_ref[0, 0] += loss_step * scale


def _make_gather_body(tpw):
    def _gather_body(table_hbm, idxg_hbm, out_hbm, idx_v, rows_v,
                     sem_g0, sem_g1, sem_s0, sem_s1):
        wid = lax.axis_index("s") * 2 + lax.axis_index("c")
        base = wid * tpw
        half = tpw // 2
        sem_g = (sem_g0, sem_g1)
        sem_s = (sem_s0, sem_s1)
        pltpu.sync_copy(idxg_hbm.at[:, pl.ds(base, tpw)], idx_v)
        scat_prev = None
        for k in range(2 * NUM_SECTIONS):
            s, h = k // 2, k % 2
            p = k % 2
            g = pltpu.async_copy(
                table_hbm.at[idx_v.at[s, pl.ds(h * half, half)]],
                rows_v.at[p], sem_g[p])
            if scat_prev is not None:
                scat_prev.wait()
            g.wait()
            scat_prev = pltpu.async_copy(
                rows_v.at[p],
                out_hbm.at[pl.ds(base + h * half, half),
                           pl.ds(s * SEC_DIM, SEC_DIM)],
                sem_s[p])
        scat_prev.wait()
    return _gather_body


CHUNKS = (18, 18, 16, 12, 6, 2)


def kernel(inputs, codebooks):
    B, T, _ = inputs.shape
    N = B * T
    x2d = inputs.reshape(N, EMBED_DIM)

    cbm2 = -2.0 * codebooks
    cn = jnp.sum(codebooks * codebooks, axis=2)
    cnc = cn[:, :, None]
    table = codebooks.reshape(NUM_SECTIONS * NUM_CENTROIDS, SEC_DIM)

    idxg_parts, loss_parts, q_parts = [], [], []
    blk0 = 0
    for cblk in CHUNKS:
        nc = cblk * TM
        tpw = nc // NW
        idxg_c, loss_c = pl.pallas_call(
            _dist_kernel,
            grid=(cblk,),
            in_specs=[
                pl.BlockSpec((TM, EMBED_DIM),
                             lambda i, b=blk0: (b + i, 0)),
                pl.BlockSpec((NUM_SECTIONS, NUM_CENTROIDS, SEC_DIM),
                             lambda i: (0, 0, 0)),
                pl.BlockSpec((NUM_SECTIONS, NUM_CENTROIDS, 1),
                             lambda i: (0, 0, 0)),
            ],
            out_specs=[
                pl.BlockSpec((NUM_SECTIONS, TM), lambda i: (0, i)),
                pl.BlockSpec(memory_space=pltpu.SMEM),
            ],
            out_shape=[
                jax.ShapeDtypeStruct((NUM_SECTIONS, nc), jnp.int32),
                jax.ShapeDtypeStruct((1, 1), jnp.float32),
            ],
        )(x2d, cbm2, cnc)
        gather = pl.kernel(
            _make_gather_body(tpw),
            out_type=jax.ShapeDtypeStruct((nc, EMBED_DIM), jnp.float32),
            mesh=plsc.VectorSubcoreMesh(core_axis_name="c",
                                        subcore_axis_name="s"),
            compiler_params=pltpu.CompilerParams(use_tc_tiling_on_sc=False),
            scratch_types=[
                pltpu.VMEM((NUM_SECTIONS, tpw), jnp.int32),
                pltpu.VMEM((2, tpw // 2, SEC_DIM), jnp.float32),
                pltpu.SemaphoreType.DMA,
                pltpu.SemaphoreType.DMA,
                pltpu.SemaphoreType.DMA,
                pltpu.SemaphoreType.DMA,
            ],
        )
        idxg_parts.append(idxg_c)
        loss_parts.append(loss_c[0, 0])
        q_parts.append(gather(table, idxg_c))
        blk0 += cblk

    q2d = jnp.concatenate(q_parts, axis=0)
    idxg = jnp.concatenate(idxg_parts, axis=1)
    loss = sum(loss_parts) / N

    quantized = q2d.reshape(B, T, EMBED_DIM)
    offs = (jnp.arange(NUM_SECTIONS, dtype=jnp.int32)
            * NUM_CENTROIDS)[:, None]
    nn_idx = (idxg - offs).reshape(NUM_SECTIONS, B, T)
    return (quantized, loss, nn_idx, table)


# speedup vs baseline: 1.0123x; 1.0123x over previous
import jax
import jax.numpy as jnp
from jax import lax
from jax.experimental import pallas as pl
from jax.experimental.pallas import tpu as pltpu
from jax.experimental.pallas import tpu_sc as plsc

NUM_SECTIONS = 4
NUM_CENTROIDS = 1024
EMBED_DIM = 256
SEC_DIM = EMBED_DIM // NUM_SECTIONS
COMMITMENT = 0.25
TM = 512
NW = 32
KAUG = 2 * SEC_DIM


def _dist_kernel(x_ref, cbm2_ref, cnc_ref, idxg_ref, loss_ref):
    i = pl.program_id(0)

    @pl.when(i == 0)
    def _():
        loss_ref[0, 0] = 0.0

    x_all = x_ref[...]
    scale = (1.0 + COMMITMENT) / (NUM_SECTIONS * SEC_DIM)
    loss_step = jnp.sum(x_all * x_all)
    for s in range(NUM_SECTIONS):
        x = x_all[:, s * SEC_DIM:(s + 1) * SEC_DIM]
        dT = jax.lax.dot_general(
            cbm2_ref[s], x, dimension_numbers=(((1,), (1,)), ((), ())),
            preferred_element_type=jnp.float32) + cnc_ref[s]
        dmin = jnp.min(dT, axis=0, keepdims=True)
        idx = jnp.argmin(dT, axis=0)
        idxg_ref[s, :] = idx + s * NUM_CENTROIDS
        loss_step += jnp.sum(dmin)
    loss_ref[0, 0] += loss_step * scale


def _make_gather_body(tpw):
    def _gather_body(table_hbm, idxg_hbm, out_hbm, idx_v, rows_v,
                     sem_g0, sem_g1, sem_s0, sem_s1):
        wid = lax.axis_index("s") * 2 + lax.axis_index("c")
        base = wid * tpw
        half = tpw // 2
        sem_g = (sem_g0, sem_g1)
        sem_s = (sem_s0, sem_s1)
        pltpu.sync_copy(idxg_hbm.at[:, pl.ds(base, tpw)], idx_v)
        scat_prev = None
        for k in range(2 * NUM_SECTIONS):
            s, h = k // 2, k % 2
            p = k % 2
            g = pltpu.async_copy(
                table_hbm.at[idx_v.at[s, pl.ds(h * half, half)]],
                rows_v.at[p], sem_g[p])
            if scat_prev is not None:
                scat_prev.wait()
            g.wait()
            scat_prev = pltpu.async_copy(
                rows_v.at[p],
                out_hbm.at[pl.ds(base + h * half, half),
                           pl.ds(s * SEC_DIM, SEC_DIM)],
                sem_s[p])
        scat_prev.wait()
    return _gather_body


CHUNKS = (18, 18, 14, 10, 8, 4)


def kernel(inputs, codebooks):
    B, T, _ = inputs.shape
    N = B * T
    x2d = inputs.reshape(N, EMBED_DIM)

    cbm2 = -2.0 * codebooks
    cn = jnp.sum(codebooks * codebooks, axis=2)
    cnc = cn[:, :, None]
    table = codebooks.reshape(NUM_SECTIONS * NUM_CENTROIDS, SEC_DIM)

    idxg_parts, loss_parts, q_parts = [], [], []
    blk0 = 0
    for cblk in CHUNKS:
        nc = cblk * TM
        tpw = nc // NW
        idxg_c, loss_c = pl.pallas_call(
            _dist_kernel,
            grid=(cblk,),
            in_specs=[
                pl.BlockSpec((TM, EMBED_DIM),
                             lambda i, b=blk0: (b + i, 0)),
                pl.BlockSpec((NUM_SECTIONS, NUM_CENTROIDS, SEC_DIM),
                             lambda i: (0, 0, 0)),
                pl.BlockSpec((NUM_SECTIONS, NUM_CENTROIDS, 1),
                             lambda i: (0, 0, 0)),
            ],
            out_specs=[
                pl.BlockSpec((NUM_SECTIONS, TM), lambda i: (0, i)),
                pl.BlockSpec(memory_space=pltpu.SMEM),
            ],
            out_shape=[
                jax.ShapeDtypeStruct((NUM_SECTIONS, nc), jnp.int32),
                jax.ShapeDtypeStruct((1, 1), jnp.float32),
            ],
        )(x2d, cbm2, cnc)
        gather = pl.kernel(
            _make_gather_body(tpw),
            out_type=jax.ShapeDtypeStruct((nc, EMBED_DIM), jnp.float32),
            mesh=plsc.VectorSubcoreMesh(core_axis_name="c",
                                        subcore_axis_name="s"),
            compiler_params=pltpu.CompilerParams(use_tc_tiling_on_sc=False),
            scratch_types=[
                pltpu.VMEM((NUM_SECTIONS, tpw), jnp.int32),
                pltpu.VMEM((2, tpw // 2, SEC_DIM), jnp.float32),
                pltpu.SemaphoreType.DMA,
                pltpu.SemaphoreType.DMA,
                pltpu.SemaphoreType.DMA,
                pltpu.SemaphoreType.DMA,
            ],
        )
        idxg_parts.append(idxg_c)
        loss_parts.append(loss_c[0, 0])
        q_parts.append(gather(table, idxg_c))
        blk0 += cblk

    q2d = jnp.concatenate(q_parts, axis=0)
    idxg = jnp.concatenate(idxg_parts, axis=1)
    loss = sum(loss_parts) / N

    quantized = q2d.reshape(B, T, EMBED_DIM)
    offs = (jnp.arange(NUM_SECTIONS, dtype=jnp.int32)
            * NUM_CENTROIDS)[:, None]
    nn_idx = (idxg - offs).reshape(NUM_SECTIONS, B, T)
    return (quantized, loss, nn_idx, table)
